# baseline (speedup 1.0000x reference)
import jax
import jax.numpy as jnp
from jax.experimental import pallas as pl
from jax.experimental.pallas import tpu as pltpu

N_OUT = 20
NC = 2
NCHUNK = 16
CM = 64


def _fwd_kernel(t_ref, x_hbm, w_ref, b_ref, o_ref, xbuf, sems):
    for i in range(NCHUNK):
        pltpu.make_async_copy(
            x_hbm.at[pl.ds(i * CM, CM), :],
            xbuf.at[pl.ds(i * CM, CM), :],
            sems.at[i],
        ).start()
    t = t_ref[0]
    cols = jax.lax.broadcasted_iota(jnp.int32, (CM, N_OUT), 1)
    keep = (cols >= t * NC) & (cols < (t + 1) * NC)
    for i in range(NCHUNK):
        pltpu.make_async_copy(
            x_hbm.at[pl.ds(i * CM, CM), :],
            xbuf.at[pl.ds(i * CM, CM), :],
            sems.at[i],
        ).wait()
        acc = jnp.dot(xbuf[i * CM:(i + 1) * CM, :], w_ref[...],
                      preferred_element_type=jnp.float32)
        o_ref[pl.ds(i * CM, CM), :] = jnp.where(
            keep, acc + b_ref[...], jnp.float32(-1.0e11))


def kernel(x, t, W, b):
    B = x.shape[0]
    xf = x.reshape(B, -1)
    K = xf.shape[1]
    t_arr = jnp.asarray(t, jnp.int32).reshape((1,))
    b2 = b.reshape(1, N_OUT)
    return pl.pallas_call(
        _fwd_kernel,
        in_specs=[
            pl.BlockSpec(memory_space=pltpu.SMEM),
            pl.BlockSpec(memory_space=pltpu.MemorySpace.HBM),
            pl.BlockSpec(memory_space=pltpu.VMEM),
            pl.BlockSpec(memory_space=pltpu.VMEM),
        ],
        out_specs=pl.BlockSpec(memory_space=pltpu.VMEM),
        out_shape=jax.ShapeDtypeStruct((B, N_OUT), jnp.float32),
        scratch_shapes=[
            pltpu.VMEM((B, K), jnp.float32),
            pltpu.SemaphoreType.DMA((NCHUNK,)),
        ],
    )(t_arr, xf, W, b2)


# speedup vs baseline: 2.7367x; 2.7367x over previous
import jax
import jax.numpy as jnp
from jax.experimental import pallas as pl
from jax.experimental.pallas import tpu as pltpu

N_OUT = 20
NC = 2
BK = 1024


def _fwd_kernel(t_ref, b01_ref, xt_ref, w_ref, o_ref, acc_ref):
    k = pl.program_id(0)
    nk = pl.num_programs(0)

    xb = xt_ref[...]
    w0 = w_ref[:, 0:1]
    w1 = w_ref[:, 1:2]
    p0 = jnp.sum(xb * w0, axis=0, keepdims=True)
    p1 = jnp.sum(xb * w1, axis=0, keepdims=True)
    part = jnp.concatenate([p0, p1], axis=0)

    @pl.when(k == 0)
    def _init():
        acc_ref[...] = part

    @pl.when(k != 0)
    def _acc():
        acc_ref[...] += part

    @pl.when(k == nk - 1)
    def _finish():
        t = t_ref[0]
        c0 = t * NC
        rows = jax.lax.broadcasted_iota(jnp.int32, (N_OUT, o_ref.shape[1]), 0)
        a0 = acc_ref[0:1, :] + b01_ref[0]
        a1 = acc_ref[1:2, :] + b01_ref[1]
        out = jnp.where(rows == c0, a0,
                        jnp.where(rows == c0 + 1, a1, jnp.float32(-1.0e11)))
        o_ref[...] = out


def kernel(x, t, W, b):
    B = x.shape[0]
    K = x.shape[1] * x.shape[2] * x.shape[3]
    xt = x.reshape(B, K).T
    t32 = jnp.asarray(t, jnp.int32)
    t_arr = t32.reshape((1,))
    w01 = jax.lax.dynamic_slice(W, (0, t32 * NC), (K, NC))
    b01 = jax.lax.dynamic_slice(b, (t32 * NC,), (NC,))
    grid = (K // BK,)
    out_t = pl.pallas_call(
        _fwd_kernel,
        grid=grid,
        in_specs=[
            pl.BlockSpec(memory_space=pltpu.SMEM),
            pl.BlockSpec(memory_space=pltpu.SMEM),
            pl.BlockSpec((BK, B), lambda k: (k, 0)),
            pl.BlockSpec((BK, NC), lambda k: (k, 0)),
        ],
        out_specs=pl.BlockSpec((N_OUT, B), lambda k: (0, 0)),
        out_shape=jax.ShapeDtypeStruct((N_OUT, B), jnp.float32),
        scratch_shapes=[pltpu.VMEM((NC, B), jnp.float32)],
        compiler_params=pltpu.CompilerParams(
            dimension_semantics=("arbitrary",),
        ),
    )(t_arr, b01, xt, w01)
    return out_t.T


# speedup vs baseline: 2.7820x; 1.0166x over previous
import jax
import jax.numpy as jnp
from jax.experimental import pallas as pl
from jax.experimental.pallas import tpu as pltpu

N_OUT = 20
NC = 2
BK = 1024
NSPLIT = 3


def _fwd_kernel(t_ref, b01_ref, *refs):
    x_refs = refs[:NSPLIT]
    w_refs = refs[NSPLIT:2 * NSPLIT]
    o_ref = refs[2 * NSPLIT]
    acc_ref = refs[2 * NSPLIT + 1]
    k = pl.program_id(0)
    nk = pl.num_programs(0)

    parts = []
    for p in range(NSPLIT):
        xb = x_refs[p][...]
        w0 = w_refs[p][:, 0:1]
        w1 = w_refs[p][:, 1:2]
        p0 = jnp.sum(xb * w0, axis=0, keepdims=True)
        p1 = jnp.sum(xb * w1, axis=0, keepdims=True)
        parts.append(jnp.concatenate([p0, p1], axis=0))
    part = parts[0]
    for q in parts[1:]:
        part = part + q

    @pl.when(k == 0)
    def _init():
        acc_ref[...] = part

    @pl.when(k != 0)
    def _acc():
        acc_ref[...] += part

    @pl.when(k == nk - 1)
    def _finish():
        t = t_ref[0]
        c0 = t * NC
        rows = jax.lax.broadcasted_iota(jnp.int32, (N_OUT, o_ref.shape[1]), 0)
        a0 = acc_ref[0:1, :] + b01_ref[0]
        a1 = acc_ref[1:2, :] + b01_ref[1]
        out = jnp.where(rows == c0, a0,
                        jnp.where(rows == c0 + 1, a1, jnp.float32(-1.0e11)))
        o_ref[...] = out


def kernel(x, t, W, b):
    B = x.shape[0]
    K = x.shape[1] * x.shape[2] * x.shape[3]
    xt = x.reshape(B, K).T
    t32 = jnp.asarray(t, jnp.int32)
    t_arr = t32.reshape((1,))
    w01 = jax.lax.dynamic_slice(W, (0, t32 * NC), (K, NC))
    b01 = jax.lax.dynamic_slice(b, (t32 * NC,), (NC,))
    nsteps = K // (BK * NSPLIT)

    def _x_spec(p):
        return pl.BlockSpec((BK, B), lambda k, p=p: (k * NSPLIT + p, 0))

    def _w_spec(p):
        return pl.BlockSpec((BK, NC), lambda k, p=p: (k * NSPLIT + p, 0))

    out_t = pl.pallas_call(
        _fwd_kernel,
        grid=(nsteps,),
        in_specs=[
            pl.BlockSpec(memory_space=pltpu.SMEM),
            pl.BlockSpec(memory_space=pltpu.SMEM),
        ] + [_x_spec(p) for p in range(NSPLIT)]
          + [_w_spec(p) for p in range(NSPLIT)],
        out_specs=pl.BlockSpec((N_OUT, B), lambda k: (0, 0)),
        out_shape=jax.ShapeDtypeStruct((N_OUT, B), jnp.float32),
        scratch_shapes=[pltpu.VMEM((NC, B), jnp.float32)],
        compiler_params=pltpu.CompilerParams(
            dimension_semantics=("arbitrary",),
        ),
    )(t_arr, b01, *([xt] * NSPLIT), *([w01] * NSPLIT))
    return out_t.T


# speedup vs baseline: 3.5988x; 1.2936x over previous
import jax
import jax.numpy as jnp
from jax.experimental import pallas as pl
from jax.experimental.pallas import tpu as pltpu

N_OUT = 20
NC = 2
BK = 1024


def _fwd_kernel(t_ref, xt_ref, wt_ref, b_ref, o_ref, acc_ref):
    k = pl.program_id(0)
    nk = pl.num_programs(0)

    part = jax.lax.dot_general(wt_ref[...], xt_ref[...],
                               (((1,), (0,)), ((), ())),
                               preferred_element_type=jnp.float32)

    @pl.when(k == 0)
    def _init():
        acc_ref[...] = part

    @pl.when(k != 0)
    def _acc():
        acc_ref[...] += part

    @pl.when(k == nk - 1)
    def _finish():
        c0 = t_ref[0] * NC
        rows = jax.lax.broadcasted_iota(jnp.int32, o_ref.shape, 0)
        keep = (rows >= c0) & (rows < c0 + NC)
        o_ref[...] = jnp.where(keep, acc_ref[...] + b_ref[...],
                               jnp.float32(-1.0e11))


def kernel(x, t, W, b):
    B = x.shape[0]
    K = x.shape[1] * x.shape[2] * x.shape[3]
    xt = x.reshape(B, K).T
    wt = W.T
    b2 = b.reshape(N_OUT, 1)
    t_arr = jnp.asarray(t, jnp.int32).reshape((1,))
    out_t = pl.pallas_call(
        _fwd_kernel,
        grid=(K // BK,),
        in_specs=[
            pl.BlockSpec(memory_space=pltpu.SMEM),
            pl.BlockSpec((BK, B), lambda k: (k, 0)),
            pl.BlockSpec((N_OUT, BK), lambda k: (0, k)),
            pl.BlockSpec((N_OUT, 1), lambda k: (0, 0)),
        ],
        out_specs=pl.BlockSpec((N_OUT, B), lambda k: (0, 0)),
        out_shape=jax.ShapeDtypeStruct((N_OUT, B), jnp.float32),
        scratch_shapes=[pltpu.VMEM((N_OUT, B), jnp.float32)],
        compiler_params=pltpu.CompilerParams(
            dimension_semantics=("arbitrary",),
        ),
    )(t_arr, xt, wt, b2)
    return out_t.T


# speedup vs baseline: 3.7343x; 1.0377x over previous
import jax
import jax.numpy as jnp
from jax.experimental import pallas as pl
from jax.experimental.pallas import tpu as pltpu

N_OUT = 20
NC = 2
BK = 2048


def _fwd_kernel(t_ref, xt_ref, wt_ref, b_ref, o_ref, acc_ref):
    k = pl.program_id(0)
    nk = pl.num_programs(0)

    part = jax.lax.dot_general(wt_ref[...], xt_ref[...],
                               (((1,), (0,)), ((), ())),
                               preferred_element_type=jnp.float32)

    @pl.when(k == 0)
    def _init():
        acc_ref[...] = part

    @pl.when(k != 0)
    def _acc():
        acc_ref[...] += part

    @pl.when(k == nk - 1)
    def _finish():
        c0 = t_ref[0] * NC
        rows = jax.lax.broadcasted_iota(jnp.int32, o_ref.shape, 0)
        keep = (rows >= c0) & (rows < c0 + NC)
        o_ref[...] = jnp.where(keep, acc_ref[...] + b_ref[...],
                               jnp.float32(-1.0e11))


def kernel(x, t, W, b):
    B = x.shape[0]
    K = x.shape[1] * x.shape[2] * x.shape[3]
    xt = x.reshape(B, K).T
    wt = W.T
    b2 = b.reshape(N_OUT, 1)
    t_arr = jnp.asarray(t, jnp.int32).reshape((1,))
    out_t = pl.pallas_call(
        _fwd_kernel,
        grid=(K // BK,),
        in_specs=[
            pl.BlockSpec(memory_space=pltpu.SMEM),
            pl.BlockSpec((BK, B), lambda k: (k, 0)),
            pl.BlockSpec((N_OUT, BK), lambda k: (0, k)),
            pl.BlockSpec((N_OUT, 1), lambda k: (0, 0)),
        ],
        out_specs=pl.BlockSpec((N_OUT, B), lambda k: (0, 0)),
        out_shape=jax.ShapeDtypeStruct((N_OUT, B), jnp.float32),
        scratch_shapes=[pltpu.VMEM((N_OUT, B), jnp.float32)],
        compiler_params=pltpu.CompilerParams(
            dimension_semantics=("arbitrary",),
        ),
    )(t_arr, xt, wt, b2)
    return out_t.T
